# Initial kernel scaffold; baseline (speedup 1.0000x reference)
import jax
import jax.numpy as jnp
from jax.experimental import pallas as pl


def kernel(node_feat, pod_feat, svc_feat, svc_src, svc_dst, in_src, in_dst, ni_src, ni_dst, W_svc, b_svc, W_in, b_in, W_ni, b_ni, Wih0, Whh0, bih0, bhh0, Wih1, Whh1, bih1, bhh1):
    raise NotImplementedError("write your pallas kernel here")


# speedup vs baseline: 1.7670x; 1.7670x over previous
import functools

import jax
import jax.numpy as jnp
from jax import lax
from jax.experimental import pallas as pl
from jax.experimental.pallas import tpu as pltpu
from jax.experimental.pallas import tpu_sc as plsc

N_NODE, N_POD, N_SVC = 10000, 30000, 4000
T, F, H = 8, 64, 64
TOTAL = N_NODE + N_POD + N_SVC

NC, NS = 2, 16
KB = 128
E_PAD_IN = 32768
E_PAD_SVC = 65536
CNT_ROWS = 44032
CNT_PER_TILE = CNT_ROWS // NS
AGG_TAB = 30208
ZC_A = 344
ZC_C = 16

_mesh = plsc.VectorSubcoreMesh(core_axis_name="c", subcore_axis_name="s",
                               num_cores=NC, num_subcores=NS)
_sc_params = pltpu.CompilerParams(use_tc_tiling_on_sc=False)


@functools.partial(
    pl.kernel,
    out_type=(jax.ShapeDtypeStruct((NC, CNT_ROWS, 16), jnp.float32),
              jax.ShapeDtypeStruct((NC, CNT_ROWS, 16), jnp.float32)),
    mesh=_mesh,
    scratch_types=[
        pltpu.VMEM_SHARED((CNT_ROWS, 16), jnp.float32),
        pltpu.VMEM_SHARED((CNT_ROWS, 16), jnp.float32),
        pltpu.VMEM((ZC_A, 16), jnp.float32),
        pltpu.VMEM((KB, 16), jnp.float32),
        pltpu.VMEM((KB,), jnp.int32),
    ],
    compiler_params=_sc_params,
)
def _sc_counts(sg_in, sg_ni, sg_svc, dg_in, dg_ni, dg_svc,
               cnt_src_out, cnt_dst_out, tab_s, tab_d, zbuf, onesbuf, idxbuf):
    c = lax.axis_index("c")
    s = lax.axis_index("s")
    wid = s * NC + c

    zero16 = jnp.zeros((16,), jnp.float32)
    e0 = jnp.where(lax.iota(jnp.int32, 16) == 0,
                   jnp.float32(1.0), jnp.float32(0.0))

    def fill_z(i, _):
        zbuf[i, :] = zero16
        return 0
    lax.fori_loop(0, ZC_A, fill_z, 0)

    def fill_o(i, _):
        onesbuf[i, :] = e0
        return 0
    lax.fori_loop(0, KB, fill_o, 0)

    r0 = s * CNT_PER_TILE

    def zero_tabs(i, _):
        pltpu.sync_copy(zbuf, tab_s.at[pl.ds(r0 + i * ZC_A, ZC_A)])
        pltpu.sync_copy(zbuf, tab_d.at[pl.ds(r0 + i * ZC_A, ZC_A)])
        return 0
    lax.fori_loop(0, CNT_PER_TILE // ZC_A, zero_tabs, 0)
    plsc.subcore_barrier()

    def scat(arr, tab, nb):
        base = wid * (nb * KB)

        def body(i, _):
            pltpu.sync_copy(arr.at[pl.ds(base + i * KB, KB)], idxbuf)
            pltpu.sync_copy(onesbuf, tab.at[idxbuf], add=True)
            return 0
        lax.fori_loop(0, nb, body, 0)

    scat(sg_in, tab_s, E_PAD_IN // (NC * NS * KB))
    scat(sg_ni, tab_s, E_PAD_IN // (NC * NS * KB))
    scat(sg_svc, tab_s, E_PAD_SVC // (NC * NS * KB))
    scat(dg_in, tab_d, E_PAD_IN // (NC * NS * KB))
    scat(dg_ni, tab_d, E_PAD_IN // (NC * NS * KB))
    scat(dg_svc, tab_d, E_PAD_SVC // (NC * NS * KB))
    plsc.subcore_barrier()

    pltpu.sync_copy(tab_s.at[pl.ds(r0, CNT_PER_TILE)],
                    cnt_src_out.at[c, pl.ds(r0, CNT_PER_TILE)])
    pltpu.sync_copy(tab_d.at[pl.ds(r0, CNT_PER_TILE)],
                    cnt_dst_out.at[c, pl.ds(r0, CNT_PER_TILE)])


@functools.partial(
    pl.kernel,
    out_type=jax.ShapeDtypeStruct((NC, T, TOTAL, F), jnp.float32),
    mesh=_mesh,
    scratch_types=[
        pltpu.VMEM_SHARED((AGG_TAB, F), jnp.float32),
        pltpu.VMEM((ZC_C, F), jnp.float32),
        pltpu.VMEM((KB, F), jnp.float32),
        pltpu.VMEM((KB,), jnp.int32),
        pltpu.VMEM((KB,), jnp.int32),
        pltpu.SemaphoreType.DMA,
    ],
    compiler_params=_sc_params,
)
def _sc_agg(xs_flat, s8_in, s8_ni, s8_svc, d_in, d_ni, d_svc,
            agg_out, tab, zbuf, rowbuf, idxs, idxd, sem):
    c = lax.axis_index("c")
    s = lax.axis_index("s")
    wid = s * NC + c

    zero16 = jnp.zeros((16,), jnp.float32)

    def fill_z(i, _):
        for k in range(F // 16):
            zbuf[i, pl.ds(k * 16, 16)] = zero16
        return 0
    lax.fori_loop(0, ZC_C, fill_z, 0)

    sections = (
        (s8_in, d_in, N_NODE, 10240, E_PAD_IN // (NC * NS * KB), 0),
        (s8_ni, d_ni, N_POD, AGG_TAB, E_PAD_IN // (NC * NS * KB), N_NODE),
        (s8_svc, d_svc, N_SVC, 4096, E_PAD_SVC // (NC * NS * KB),
         N_NODE + N_POD),
    )

    for (srcarr, dstarr, n_dst, tabrows, nb, gbase) in sections:
        zpt = tabrows // NS
        cpt = n_dst // NS
        ebase = wid * (nb * KB)

        def per_t(t, _, srcarr=srcarr, dstarr=dstarr, n_dst=n_dst,
                  zpt=zpt, cpt=cpt, nb=nb, gbase=gbase, ebase=ebase):
            def z(i, _):
                pltpu.sync_copy(zbuf, tab.at[pl.ds(s * zpt + i * ZC_C, ZC_C)])
                return 0
            lax.fori_loop(0, zpt // ZC_C, z, 0)
            plsc.subcore_barrier()

            def b(i, _):
                off = ebase + i * KB
                pltpu.sync_copy(srcarr.at[pl.ds(off, KB)], idxs)
                for j in range(KB // 16):
                    idxs[pl.ds(j * 16, 16)] = idxs[pl.ds(j * 16, 16)] + t
                pltpu.async_copy(xs_flat.at[idxs], rowbuf, sem).wait()
                pltpu.sync_copy(dstarr.at[pl.ds(off, KB)], idxd)
                pltpu.sync_copy(rowbuf, tab.at[idxd], add=True)
                return 0
            lax.fori_loop(0, nb, b, 0)
            plsc.subcore_barrier()

            nco = cpt // 125
            def co(i, _):
                r = s * cpt + i * 125
                pltpu.sync_copy(tab.at[pl.ds(r, 125)],
                                agg_out.at[c, t, pl.ds(gbase + r, 125)])
                return 0
            lax.fori_loop(0, nco, co, 0)
            plsc.subcore_barrier()
            return 0

        lax.fori_loop(0, T, per_t, 0)


def _prescale_body(x_ref, cnt_ref, out_ref):
    deg = cnt_ref[0, :, 0] + cnt_ref[1, :, 0]
    nrm = lax.rsqrt(jnp.maximum(deg, 1.0))
    out_ref[...] = x_ref[...] * nrm[:, None]


def _prescale(xcat, cnt_src, R=400):
    n = xcat.shape[0]
    return pl.pallas_call(
        _prescale_body,
        grid=(n // R,),
        in_specs=[
            pl.BlockSpec((R, T * F), lambda i: (i, 0)),
            pl.BlockSpec((NC, R, 16), lambda i: (0, i, 0)),
        ],
        out_specs=pl.BlockSpec((R, T * F), lambda i: (i, 0)),
        out_shape=jax.ShapeDtypeStruct((n, T * F), jnp.float32),
    )(xcat, cnt_src)


def _conv_lstm_body(agg_ref, cnt_ref, W_ref, b_ref,
                    wih0_ref, whh0_ref, bias0_ref,
                    wih1_ref, whh1_ref, bias1_ref,
                    out_ref):
    R = agg_ref.shape[2]
    deg = cnt_ref[0, :, 0] + cnt_ref[1, :, 0]
    nrm = lax.rsqrt(jnp.maximum(deg, 1.0))

    def lrelu(v):
        return jnp.where(v > 0, v, 0.01 * v)

    xs = []
    for t in range(T):
        st = agg_ref[0, t] + agg_ref[1, t]
        y = jnp.dot(st * nrm[:, None], W_ref[t],
                    preferred_element_type=jnp.float32) + b_ref[t]
        xs.append(lrelu(y))

    def lstm(x_list, wihT, whhT, bias):
        h = jnp.zeros((R, H), jnp.float32)
        cc = jnp.zeros((R, H), jnp.float32)
        outs = []
        for t in range(T):
            g = (jnp.dot(x_list[t], wihT, preferred_element_type=jnp.float32)
                 + jnp.dot(h, whhT, preferred_element_type=jnp.float32)
                 + bias)
            i = jax.nn.sigmoid(g[:, 0 * H:1 * H])
            f = jax.nn.sigmoid(g[:, 1 * H:2 * H])
            gg = jnp.tanh(g[:, 2 * H:3 * H])
            o = jax.nn.sigmoid(g[:, 3 * H:4 * H])
            cc = f * cc + i * gg
            h = o * jnp.tanh(cc)
            outs.append(h)
        return outs

    h1 = lstm(xs, wih0_ref[...], whh0_ref[...], bias0_ref[...])
    h2 = lstm(h1, wih1_ref[...], whh1_ref[...], bias1_ref[...])
    out_ref[...] = jnp.stack(h2, axis=1)


def _conv_lstm(agg, cnt, W, b, wih0T, whh0T, bias0, wih1T, whh1T, bias1, R):
    n = agg.shape[2]
    return pl.pallas_call(
        _conv_lstm_body,
        grid=(n // R,),
        in_specs=[
            pl.BlockSpec((NC, T, R, F), lambda i: (0, 0, i, 0)),
            pl.BlockSpec((NC, R, 16), lambda i: (0, i, 0)),
            pl.BlockSpec((T, F, H), lambda i: (0, 0, 0)),
            pl.BlockSpec((T, 1, H), lambda i: (0, 0, 0)),
            pl.BlockSpec((H, 4 * H), lambda i: (0, 0)),
            pl.BlockSpec((H, 4 * H), lambda i: (0, 0)),
            pl.BlockSpec((1, 4 * H), lambda i: (0, 0)),
            pl.BlockSpec((H, 4 * H), lambda i: (0, 0)),
            pl.BlockSpec((H, 4 * H), lambda i: (0, 0)),
            pl.BlockSpec((1, 4 * H), lambda i: (0, 0)),
        ],
        out_specs=pl.BlockSpec((R, T, H), lambda i: (i, 0, 0)),
        out_shape=jax.ShapeDtypeStruct((n, T, H), jnp.float32),
    )(agg, cnt, W, b.reshape(T, 1, H), wih0T, whh0T, bias0.reshape(1, 4 * H),
      wih1T, whh1T, bias1.reshape(1, 4 * H))


def _padto(a, n, fill):
    return jnp.concatenate(
        [a.astype(jnp.int32), jnp.full((n - a.shape[0],), fill, jnp.int32)])


def kernel(node_feat, pod_feat, svc_feat, svc_src, svc_dst, in_src, in_dst,
           ni_src, ni_dst, W_svc, b_svc, W_in, b_in, W_ni, b_ni,
           Wih0, Whh0, bih0, bhh0, Wih1, Whh1, bih1, bhh1):
    xcat = jnp.concatenate([node_feat.reshape(N_NODE, T * F),
                            pod_feat.reshape(N_POD, T * F),
                            svc_feat.reshape(N_SVC, T * F)], axis=0)

    sg_in = _padto(in_src + N_NODE, E_PAD_IN, TOTAL)
    sg_ni = _padto(ni_src, E_PAD_IN, TOTAL)
    sg_svc = _padto(svc_src + N_NODE + N_POD, E_PAD_SVC, TOTAL)
    dg_in = _padto(in_dst, E_PAD_IN, TOTAL)
    dg_ni = _padto(ni_dst + N_NODE, E_PAD_IN, TOTAL)
    dg_svc = _padto(svc_dst + N_NODE + N_POD, E_PAD_SVC, TOTAL)

    s8_in = _padto((in_src + N_NODE) * T, E_PAD_IN, 0)
    s8_ni = _padto(ni_src * T, E_PAD_IN, 0)
    s8_svc = _padto((svc_src + N_NODE + N_POD) * T, E_PAD_SVC, 0)
    d_in = _padto(in_dst, E_PAD_IN, N_NODE)
    d_ni = _padto(ni_dst, E_PAD_IN, N_POD)
    d_svc = _padto(svc_dst, E_PAD_SVC, N_SVC)

    cnt_src, cnt_dst = _sc_counts(sg_in, sg_ni, sg_svc, dg_in, dg_ni, dg_svc)
    xs = _prescale(xcat, cnt_src[:, :TOTAL])
    agg = _sc_agg(xs.reshape(TOTAL * T, F),
                  s8_in, s8_ni, s8_svc, d_in, d_ni, d_svc)

    wih0T, whh0T = Wih0.T, Whh0.T
    wih1T, whh1T = Wih1.T, Whh1.T
    bias0 = bih0 + bhh0
    bias1 = bih1 + bhh1

    def run(lo, hi, W, b, R):
        return _conv_lstm(agg[:, :, lo:hi], cnt_dst[:, lo:hi], W, b,
                          wih0T, whh0T, bias0, wih1T, whh1T, bias1, R)

    out_node = run(0, N_NODE, W_in, b_in, 400)
    out_pod = run(N_NODE, N_NODE + N_POD, W_ni, b_ni, 400)
    out_svc = run(N_NODE + N_POD, TOTAL, W_svc, b_svc, 400)
    return jnp.concatenate([out_node, out_pod, out_svc], axis=0)


# speedup vs baseline: 2.1964x; 1.2430x over previous
import functools

import jax
import jax.numpy as jnp
from jax import lax
from jax.experimental import pallas as pl
from jax.experimental.pallas import tpu as pltpu
from jax.experimental.pallas import tpu_sc as plsc

N_NODE, N_POD, N_SVC = 10000, 30000, 4000
T, F, H = 8, 64, 64
TOTAL = N_NODE + N_POD + N_SVC

NC, NS = 2, 16
KB = 128
KBC = 64
E_PAD_IN = 32768
E_PAD_SVC = 65536
E_EXTRA = KBC
CNT_ROWS = 44032
CNT_PER_TILE = CNT_ROWS // NS
AGG_TAB = 30016
ZROWS = 1888
ZC_A = 344

_mesh = plsc.VectorSubcoreMesh(core_axis_name="c", subcore_axis_name="s",
                               num_cores=NC, num_subcores=NS)
_sc_params = pltpu.CompilerParams(use_tc_tiling_on_sc=False)


@functools.partial(
    pl.kernel,
    out_type=(jax.ShapeDtypeStruct((NC, CNT_ROWS, 16), jnp.float32),
              jax.ShapeDtypeStruct((NC, CNT_ROWS, 16), jnp.float32)),
    mesh=_mesh,
    scratch_types=[
        pltpu.VMEM_SHARED((CNT_ROWS, 16), jnp.float32),
        pltpu.VMEM_SHARED((CNT_ROWS, 16), jnp.float32),
        pltpu.VMEM((ZC_A, 16), jnp.float32),
        pltpu.VMEM((KB, 16), jnp.float32),
        pltpu.VMEM((KB,), jnp.int32),
    ],
    compiler_params=_sc_params,
)
def _sc_counts(sg_in, sg_ni, sg_svc, dg_in, dg_ni, dg_svc,
               cnt_src_out, cnt_dst_out, tab_s, tab_d, zbuf, onesbuf, idxbuf):
    c = lax.axis_index("c")
    s = lax.axis_index("s")
    wid = s * NC + c

    zero16 = jnp.zeros((16,), jnp.float32)
    e0 = jnp.where(lax.iota(jnp.int32, 16) == 0,
                   jnp.float32(1.0), jnp.float32(0.0))

    def fill_z(i, _):
        zbuf[i, :] = zero16
        return 0
    lax.fori_loop(0, ZC_A, fill_z, 0)

    def fill_o(i, _):
        onesbuf[i, :] = e0
        return 0
    lax.fori_loop(0, KB, fill_o, 0)

    r0 = s * CNT_PER_TILE

    def zero_tabs(i, _):
        pltpu.sync_copy(zbuf, tab_s.at[pl.ds(r0 + i * ZC_A, ZC_A)])
        pltpu.sync_copy(zbuf, tab_d.at[pl.ds(r0 + i * ZC_A, ZC_A)])
        return 0
    lax.fori_loop(0, CNT_PER_TILE // ZC_A, zero_tabs, 0)
    plsc.subcore_barrier()

    def scat(arr, tab, nb):
        base = wid * (nb * KB)

        def body(i, _):
            pltpu.sync_copy(arr.at[pl.ds(base + i * KB, KB)], idxbuf)
            pltpu.sync_copy(onesbuf, tab.at[idxbuf], add=True)
            return 0
        lax.fori_loop(0, nb, body, 0)

    scat(sg_in, tab_s, E_PAD_IN // (NC * NS * KB))
    scat(sg_ni, tab_s, E_PAD_IN // (NC * NS * KB))
    scat(sg_svc, tab_s, E_PAD_SVC // (NC * NS * KB))
    scat(dg_in, tab_d, E_PAD_IN // (NC * NS * KB))
    scat(dg_ni, tab_d, E_PAD_IN // (NC * NS * KB))
    scat(dg_svc, tab_d, E_PAD_SVC // (NC * NS * KB))
    plsc.subcore_barrier()

    pltpu.sync_copy(tab_s.at[pl.ds(r0, CNT_PER_TILE)],
                    cnt_src_out.at[c, pl.ds(r0, CNT_PER_TILE)])
    pltpu.sync_copy(tab_d.at[pl.ds(r0, CNT_PER_TILE)],
                    cnt_dst_out.at[c, pl.ds(r0, CNT_PER_TILE)])


@functools.partial(
    pl.kernel,
    out_type=jax.ShapeDtypeStruct((T, TOTAL, F), jnp.float32),
    mesh=_mesh,
    scratch_types=[
        pltpu.VMEM_SHARED((AGG_TAB, F), jnp.float32),
        pltpu.VMEM((KBC, F), jnp.float32),
        pltpu.VMEM((KBC,), jnp.int32),
        pltpu.VMEM((KBC,), jnp.int32),
        pltpu.SemaphoreType.DMA,
        pltpu.VMEM((KBC, F), jnp.float32),
        pltpu.VMEM((KBC,), jnp.int32),
        pltpu.VMEM((KBC,), jnp.int32),
        pltpu.SemaphoreType.DMA,
    ],
    compiler_params=_sc_params,
)
def _sc_agg(xs_flat, s8_in, s8_ni, s8_svc, d_in, d_ni, d_svc, zhbm,
            agg_out, tab, rowA, idxsA, idxdA, semA, rowB, idxsB, idxdB, semB):
    c = lax.axis_index("c")
    s = lax.axis_index("s")
    bufs = ((rowA, idxsA, idxdA, semA), (rowB, idxsB, idxdB, semB))

    sections = (
        (s8_in, d_in, N_NODE, 10240, E_PAD_IN // (NS * KBC), 0, 1),
        (s8_ni, d_ni, N_POD, AGG_TAB, E_PAD_IN // (NS * KBC), N_NODE, 1),
        (s8_svc, d_svc, N_SVC, 4096, E_PAD_SVC // (NS * KBC),
         N_NODE + N_POD, 0),
    )

    for (srcarr, dstarr, n_dst, tabrows, nb, gbase, core) in sections:
        zpt = tabrows // NS
        cpt = n_dst // NS
        ebase = s * (nb * KBC)

        def start(buf, off, t, srcarr=srcarr, dstarr=dstarr):
            rb, ixs, ixd, sm = buf
            pltpu.sync_copy(srcarr.at[pl.ds(off, KBC)], ixs)
            for j in range(KBC // 16):
                ixs[pl.ds(j * 16, 16)] = ixs[pl.ds(j * 16, 16)] + t
            pltpu.async_copy(xs_flat.at[ixs], rb, sm)
            pltpu.sync_copy(dstarr.at[pl.ds(off, KBC)], ixd)

        def finish(buf):
            rb, ixs, ixd, sm = buf
            pltpu.make_async_copy(xs_flat.at[ixs], rb, sm).wait()
            pltpu.sync_copy(rb, tab.at[ixd], add=True)

        def drain(buf):
            rb, ixs, ixd, sm = buf
            pltpu.make_async_copy(xs_flat.at[ixs], rb, sm).wait()

        def per_t(t, _, zpt=zpt, cpt=cpt, nb=nb, gbase=gbase, ebase=ebase,
                  start=start, finish=finish, drain=drain):
            pltpu.sync_copy(zhbm.at[pl.ds(0, zpt)],
                            tab.at[pl.ds(s * zpt, zpt)])
            plsc.subcore_barrier()

            start(bufs[0], ebase, t)

            def body(g, _):
                start(bufs[1], ebase + (2 * g + 1) * KBC, t)
                finish(bufs[0])
                start(bufs[0], ebase + (2 * g + 2) * KBC, t)
                finish(bufs[1])
                return 0
            lax.fori_loop(0, nb // 2, body, 0)
            drain(bufs[0])
            plsc.subcore_barrier()

            pltpu.sync_copy(tab.at[pl.ds(s * cpt, cpt)],
                            agg_out.at[t, pl.ds(gbase + s * cpt, cpt)])
            plsc.subcore_barrier()
            return 0

        @pl.when(c == core)
        def _run():
            lax.fori_loop(0, T, per_t, 0)


def _prescale_body(x_ref, cnt_ref, out_ref):
    deg = cnt_ref[0, :, 0] + cnt_ref[1, :, 0]
    nrm = lax.rsqrt(jnp.maximum(deg, 1.0))
    out_ref[...] = x_ref[...] * nrm[:, None]


def _prescale(xcat, cnt_src, R=400):
    n = xcat.shape[0]
    return pl.pallas_call(
        _prescale_body,
        grid=(n // R,),
        in_specs=[
            pl.BlockSpec((R, T * F), lambda i: (i, 0)),
            pl.BlockSpec((NC, R, 16), lambda i: (0, i, 0)),
        ],
        out_specs=pl.BlockSpec((R, T * F), lambda i: (i, 0)),
        out_shape=jax.ShapeDtypeStruct((n, T * F), jnp.float32),
    )(xcat, cnt_src)


def _conv_lstm_body(agg_ref, cnt_ref, W_ref, b_ref,
                    wc0_ref, bias0_ref, wc1_ref, bias1_ref,
                    out_ref):
    R = agg_ref.shape[1]
    deg = cnt_ref[0, :, 0] + cnt_ref[1, :, 0]
    nrm = lax.rsqrt(jnp.maximum(deg, 1.0))

    def lrelu(v):
        return jnp.where(v > 0, v, 0.01 * v)

    xs = []
    for t in range(T):
        y = jnp.dot(agg_ref[t] * nrm[:, None], W_ref[t],
                    preferred_element_type=jnp.float32) + b_ref[t]
        xs.append(lrelu(y))

    def lstm(x_list, wc, bias):
        h = jnp.zeros((R, H), jnp.float32)
        cc = jnp.zeros((R, H), jnp.float32)
        outs = []
        for t in range(T):
            xh = jnp.concatenate([x_list[t], h], axis=1)
            g = jnp.dot(xh, wc, preferred_element_type=jnp.float32) + bias
            i = jax.nn.sigmoid(g[:, 0 * H:1 * H])
            f = jax.nn.sigmoid(g[:, 1 * H:2 * H])
            gg = jnp.tanh(g[:, 2 * H:3 * H])
            o = jax.nn.sigmoid(g[:, 3 * H:4 * H])
            cc = f * cc + i * gg
            h = o * jnp.tanh(cc)
            outs.append(h)
        return outs

    h1 = lstm(xs, wc0_ref[...], bias0_ref[...])
    h2 = lstm(h1, wc1_ref[...], bias1_ref[...])
    out_ref[...] = jnp.stack(h2, axis=1)


def _conv_lstm(agg, cnt, W, b, wc0, bias0, wc1, bias1, R):
    n = agg.shape[1]
    return pl.pallas_call(
        _conv_lstm_body,
        grid=(n // R,),
        in_specs=[
            pl.BlockSpec((T, R, F), lambda i: (0, i, 0)),
            pl.BlockSpec((NC, R, 16), lambda i: (0, i, 0)),
            pl.BlockSpec((T, F, H), lambda i: (0, 0, 0)),
            pl.BlockSpec((T, 1, H), lambda i: (0, 0, 0)),
            pl.BlockSpec((2 * H, 4 * H), lambda i: (0, 0)),
            pl.BlockSpec((1, 4 * H), lambda i: (0, 0)),
            pl.BlockSpec((2 * H, 4 * H), lambda i: (0, 0)),
            pl.BlockSpec((1, 4 * H), lambda i: (0, 0)),
        ],
        out_specs=pl.BlockSpec((R, T, H), lambda i: (i, 0, 0)),
        out_shape=jax.ShapeDtypeStruct((n, T, H), jnp.float32),
    )(agg, cnt, W, b.reshape(T, 1, H), wc0, bias0.reshape(1, 4 * H),
      wc1, bias1.reshape(1, 4 * H))


def _padto(a, n, fill):
    return jnp.concatenate(
        [a.astype(jnp.int32), jnp.full((n - a.shape[0],), fill, jnp.int32)])


def kernel(node_feat, pod_feat, svc_feat, svc_src, svc_dst, in_src, in_dst,
           ni_src, ni_dst, W_svc, b_svc, W_in, b_in, W_ni, b_ni,
           Wih0, Whh0, bih0, bhh0, Wih1, Whh1, bih1, bhh1):
    xcat = jnp.concatenate([node_feat.reshape(N_NODE, T * F),
                            pod_feat.reshape(N_POD, T * F),
                            svc_feat.reshape(N_SVC, T * F)], axis=0)

    sg_in = _padto(in_src + N_NODE, E_PAD_IN, TOTAL)
    sg_ni = _padto(ni_src, E_PAD_IN, TOTAL)
    sg_svc = _padto(svc_src + N_NODE + N_POD, E_PAD_SVC, TOTAL)
    dg_in = _padto(in_dst, E_PAD_IN, TOTAL)
    dg_ni = _padto(ni_dst + N_NODE, E_PAD_IN, TOTAL)
    dg_svc = _padto(svc_dst + N_NODE + N_POD, E_PAD_SVC, TOTAL)

    s8_in = _padto((in_src + N_NODE) * T, E_PAD_IN + E_EXTRA, 0)
    s8_ni = _padto(ni_src * T, E_PAD_IN + E_EXTRA, 0)
    s8_svc = _padto((svc_src + N_NODE + N_POD) * T, E_PAD_SVC + E_EXTRA, 0)
    d_in = _padto(in_dst, E_PAD_IN + E_EXTRA, N_NODE)
    d_ni = _padto(ni_dst, E_PAD_IN + E_EXTRA, N_POD)
    d_svc = _padto(svc_dst, E_PAD_SVC + E_EXTRA, N_SVC)
    zhbm = jnp.zeros((ZROWS, F), jnp.float32)

    cnt_src, cnt_dst = _sc_counts(sg_in, sg_ni, sg_svc, dg_in, dg_ni, dg_svc)
    xs = _prescale(xcat, cnt_src[:, :TOTAL])
    agg = _sc_agg(xs.reshape(TOTAL * T, F),
                  s8_in, s8_ni, s8_svc, d_in, d_ni, d_svc, zhbm)

    bias0 = bih0 + bhh0
    bias1 = bih1 + bhh1
    wc0 = jnp.concatenate([Wih0.T, Whh0.T], axis=0)
    wc1 = jnp.concatenate([Wih1.T, Whh1.T], axis=0)

    def run(lo, hi, W, b, R):
        return _conv_lstm(agg[:, lo:hi], cnt_dst[:, lo:hi], W, b,
                          wc0, bias0, wc1, bias1, R)

    out_node = run(0, N_NODE, W_in, b_in, 1000)
    out_pod = run(N_NODE, N_NODE + N_POD, W_ni, b_ni, 1000)
    out_svc = run(N_NODE + N_POD, TOTAL, W_svc, b_svc, 1000)
    return jnp.concatenate([out_node, out_pod, out_svc], axis=0)


# speedup vs baseline: 2.3540x; 1.0718x over previous
import functools

import jax
import jax.numpy as jnp
from jax import lax
from jax.experimental import pallas as pl
from jax.experimental.pallas import tpu as pltpu
from jax.experimental.pallas import tpu_sc as plsc

N_NODE, N_POD, N_SVC = 10000, 30000, 4000
T, F, H = 8, 64, 64
TOTAL = N_NODE + N_POD + N_SVC

NC, NS = 2, 16
KB = 128
KBC = 64
E_PAD_IN = 32768
E_PAD_SVC = 65536
E_EXTRA = KBC
CNT_ROWS = 44032
CNT_PER_TILE = CNT_ROWS // NS
AGG_TAB = 30016
ZROWS = 1888
ZC_A = 344

_mesh = plsc.VectorSubcoreMesh(core_axis_name="c", subcore_axis_name="s",
                               num_cores=NC, num_subcores=NS)
_sc_params = pltpu.CompilerParams(use_tc_tiling_on_sc=False)


@functools.partial(
    pl.kernel,
    out_type=(jax.ShapeDtypeStruct((NC, CNT_ROWS, 16), jnp.float32),
              jax.ShapeDtypeStruct((NC, CNT_ROWS, 16), jnp.float32)),
    mesh=_mesh,
    scratch_types=[
        pltpu.VMEM_SHARED((CNT_ROWS, 16), jnp.float32),
        pltpu.VMEM_SHARED((CNT_ROWS, 16), jnp.float32),
        pltpu.VMEM((ZC_A, 16), jnp.float32),
        pltpu.VMEM((KB, 16), jnp.float32),
        pltpu.VMEM((KB,), jnp.int32),
    ],
    compiler_params=_sc_params,
)
def _sc_counts(sg_in, sg_ni, sg_svc, dg_in, dg_ni, dg_svc,
               cnt_src_out, cnt_dst_out, tab_s, tab_d, zbuf, onesbuf, idxbuf):
    c = lax.axis_index("c")
    s = lax.axis_index("s")
    wid = s * NC + c

    zero16 = jnp.zeros((16,), jnp.float32)
    e0 = jnp.where(lax.iota(jnp.int32, 16) == 0,
                   jnp.float32(1.0), jnp.float32(0.0))

    def fill_z(i, _):
        zbuf[i, :] = zero16
        return 0
    lax.fori_loop(0, ZC_A, fill_z, 0)

    def fill_o(i, _):
        onesbuf[i, :] = e0
        return 0
    lax.fori_loop(0, KB, fill_o, 0)

    r0 = s * CNT_PER_TILE

    def zero_tabs(i, _):
        pltpu.sync_copy(zbuf, tab_s.at[pl.ds(r0 + i * ZC_A, ZC_A)])
        pltpu.sync_copy(zbuf, tab_d.at[pl.ds(r0 + i * ZC_A, ZC_A)])
        return 0
    lax.fori_loop(0, CNT_PER_TILE // ZC_A, zero_tabs, 0)
    plsc.subcore_barrier()

    def scat(arr, tab, nb):
        base = wid * (nb * KB)

        def body(i, _):
            pltpu.sync_copy(arr.at[pl.ds(base + i * KB, KB)], idxbuf)
            pltpu.sync_copy(onesbuf, tab.at[idxbuf], add=True)
            return 0
        lax.fori_loop(0, nb, body, 0)

    scat(sg_in, tab_s, E_PAD_IN // (NC * NS * KB))
    scat(sg_ni, tab_s, E_PAD_IN // (NC * NS * KB))
    scat(sg_svc, tab_s, E_PAD_SVC // (NC * NS * KB))
    scat(dg_in, tab_d, E_PAD_IN // (NC * NS * KB))
    scat(dg_ni, tab_d, E_PAD_IN // (NC * NS * KB))
    scat(dg_svc, tab_d, E_PAD_SVC // (NC * NS * KB))
    plsc.subcore_barrier()

    pltpu.sync_copy(tab_s.at[pl.ds(r0, CNT_PER_TILE)],
                    cnt_src_out.at[c, pl.ds(r0, CNT_PER_TILE)])
    pltpu.sync_copy(tab_d.at[pl.ds(r0, CNT_PER_TILE)],
                    cnt_dst_out.at[c, pl.ds(r0, CNT_PER_TILE)])


@functools.partial(
    pl.kernel,
    out_type=jax.ShapeDtypeStruct((T, TOTAL, F), jnp.float32),
    mesh=_mesh,
    scratch_types=[
        pltpu.VMEM_SHARED((AGG_TAB, F), jnp.float32),
        pltpu.VMEM((KBC, F), jnp.float32),
        pltpu.VMEM((KBC,), jnp.int32),
        pltpu.VMEM((KBC,), jnp.int32),
        pltpu.SemaphoreType.DMA,
        pltpu.VMEM((KBC, F), jnp.float32),
        pltpu.VMEM((KBC,), jnp.int32),
        pltpu.VMEM((KBC,), jnp.int32),
        pltpu.SemaphoreType.DMA,
    ],
    compiler_params=_sc_params,
)
def _sc_agg(xs_flat, s8_in, s8_ni, s8_svc, d_in, d_ni, d_svc, zhbm,
            agg_out, tab, rowA, idxsA, idxdA, semA, rowB, idxsB, idxdB, semB):
    c = lax.axis_index("c")
    s = lax.axis_index("s")
    bufs = ((rowA, idxsA, idxdA, semA), (rowB, idxsB, idxdB, semB))

    sections = (
        (s8_in, d_in, N_NODE, 10240, E_PAD_IN // (NS * KBC), 0, 1, 0, T),
        (s8_ni, d_ni, N_POD, AGG_TAB, E_PAD_IN // (NS * KBC), N_NODE,
         0, 0, 3),
        (s8_ni, d_ni, N_POD, AGG_TAB, E_PAD_IN // (NS * KBC), N_NODE,
         1, 3, T),
        (s8_svc, d_svc, N_SVC, 4096, E_PAD_SVC // (NS * KBC),
         N_NODE + N_POD, 0, 0, T),
    )

    for (srcarr, dstarr, n_dst, tabrows, nb, gbase, core,
         t_lo, t_hi) in sections:
        zpt = tabrows // NS
        cpt = n_dst // NS
        ebase = s * (nb * KBC)

        def start(buf, off, t, srcarr=srcarr, dstarr=dstarr):
            rb, ixs, ixd, sm = buf
            pltpu.sync_copy(srcarr.at[pl.ds(off, KBC)], ixs)
            for j in range(KBC // 16):
                ixs[pl.ds(j * 16, 16)] = ixs[pl.ds(j * 16, 16)] + t
            pltpu.async_copy(xs_flat.at[ixs], rb, sm)
            pltpu.sync_copy(dstarr.at[pl.ds(off, KBC)], ixd)

        def finish(buf):
            rb, ixs, ixd, sm = buf
            pltpu.make_async_copy(xs_flat.at[ixs], rb, sm).wait()
            pltpu.sync_copy(rb, tab.at[ixd], add=True)

        def drain(buf):
            rb, ixs, ixd, sm = buf
            pltpu.make_async_copy(xs_flat.at[ixs], rb, sm).wait()

        def per_t(t, _, zpt=zpt, cpt=cpt, nb=nb, gbase=gbase, ebase=ebase,
                  start=start, finish=finish, drain=drain):
            pltpu.sync_copy(zhbm.at[pl.ds(0, zpt)],
                            tab.at[pl.ds(s * zpt, zpt)])
            plsc.subcore_barrier()

            start(bufs[0], ebase, t)

            def body(g, _):
                start(bufs[1], ebase + (2 * g + 1) * KBC, t)
                finish(bufs[0])
                start(bufs[0], ebase + (2 * g + 2) * KBC, t)
                finish(bufs[1])
                return 0
            lax.fori_loop(0, nb // 2, body, 0)
            drain(bufs[0])
            plsc.subcore_barrier()

            pltpu.sync_copy(tab.at[pl.ds(s * cpt, cpt)],
                            agg_out.at[t, pl.ds(gbase + s * cpt, cpt)])
            plsc.subcore_barrier()
            return 0

        @pl.when(c == core)
        def _run(per_t=per_t, t_lo=t_lo, t_hi=t_hi):
            lax.fori_loop(t_lo, t_hi, per_t, 0)


def _prescale_body(x_ref, cnt_ref, out_ref):
    deg = cnt_ref[0, :, 0] + cnt_ref[1, :, 0]
    nrm = lax.rsqrt(jnp.maximum(deg, 1.0))
    out_ref[...] = x_ref[...] * nrm[:, None]


def _prescale(xcat, cnt_src, R=400):
    n = xcat.shape[0]
    return pl.pallas_call(
        _prescale_body,
        grid=(n // R,),
        in_specs=[
            pl.BlockSpec((R, T * F), lambda i: (i, 0)),
            pl.BlockSpec((NC, R, 16), lambda i: (0, i, 0)),
        ],
        out_specs=pl.BlockSpec((R, T * F), lambda i: (i, 0)),
        out_shape=jax.ShapeDtypeStruct((n, T * F), jnp.float32),
    )(xcat, cnt_src)


def _conv_lstm_body(agg_ref, cnt_ref, W_ref, b_ref,
                    wc0_ref, bias0_ref, wc1_ref, bias1_ref,
                    out_ref):
    R = agg_ref.shape[1]
    deg = cnt_ref[0, :, 0] + cnt_ref[1, :, 0]
    nrm = lax.rsqrt(jnp.maximum(deg, 1.0))

    def lrelu(v):
        return jnp.where(v > 0, v, 0.01 * v)

    bf = jnp.bfloat16
    xs = []
    for t in range(T):
        y = jnp.dot((agg_ref[t] * nrm[:, None]).astype(bf),
                    W_ref[t].astype(bf),
                    preferred_element_type=jnp.float32) + b_ref[t]
        xs.append(lrelu(y))

    def lstm(x_list, wc, bias):
        wcb = wc.astype(bf)
        h = jnp.zeros((R, H), jnp.float32)
        cc = jnp.zeros((R, H), jnp.float32)
        outs = []
        for t in range(T):
            xh = jnp.concatenate([x_list[t], h], axis=1).astype(bf)
            g = jnp.dot(xh, wcb, preferred_element_type=jnp.float32) + bias
            i = jax.nn.sigmoid(g[:, 0 * H:1 * H])
            f = jax.nn.sigmoid(g[:, 1 * H:2 * H])
            gg = jnp.tanh(g[:, 2 * H:3 * H])
            o = jax.nn.sigmoid(g[:, 3 * H:4 * H])
            cc = f * cc + i * gg
            h = o * jnp.tanh(cc)
            outs.append(h)
        return outs

    h1 = lstm(xs, wc0_ref[...], bias0_ref[...])
    h2 = lstm(h1, wc1_ref[...], bias1_ref[...])
    out_ref[...] = jnp.stack(h2, axis=1)


def _conv_lstm(agg, cnt, W, b, wc0, bias0, wc1, bias1, R):
    n = agg.shape[1]
    return pl.pallas_call(
        _conv_lstm_body,
        grid=(n // R,),
        in_specs=[
            pl.BlockSpec((T, R, F), lambda i: (0, i, 0)),
            pl.BlockSpec((NC, R, 16), lambda i: (0, i, 0)),
            pl.BlockSpec((T, F, H), lambda i: (0, 0, 0)),
            pl.BlockSpec((T, 1, H), lambda i: (0, 0, 0)),
            pl.BlockSpec((2 * H, 4 * H), lambda i: (0, 0)),
            pl.BlockSpec((1, 4 * H), lambda i: (0, 0)),
            pl.BlockSpec((2 * H, 4 * H), lambda i: (0, 0)),
            pl.BlockSpec((1, 4 * H), lambda i: (0, 0)),
        ],
        out_specs=pl.BlockSpec((R, T, H), lambda i: (i, 0, 0)),
        out_shape=jax.ShapeDtypeStruct((n, T, H), jnp.float32),
    )(agg, cnt, W, b.reshape(T, 1, H), wc0, bias0.reshape(1, 4 * H),
      wc1, bias1.reshape(1, 4 * H))


def _padto(a, n, fill):
    return jnp.concatenate(
        [a.astype(jnp.int32), jnp.full((n - a.shape[0],), fill, jnp.int32)])


def kernel(node_feat, pod_feat, svc_feat, svc_src, svc_dst, in_src, in_dst,
           ni_src, ni_dst, W_svc, b_svc, W_in, b_in, W_ni, b_ni,
           Wih0, Whh0, bih0, bhh0, Wih1, Whh1, bih1, bhh1):
    xcat = jnp.concatenate([node_feat.reshape(N_NODE, T * F),
                            pod_feat.reshape(N_POD, T * F),
                            svc_feat.reshape(N_SVC, T * F)], axis=0)

    sg_in = _padto(in_src + N_NODE, E_PAD_IN, TOTAL)
    sg_ni = _padto(ni_src, E_PAD_IN, TOTAL)
    sg_svc = _padto(svc_src + N_NODE + N_POD, E_PAD_SVC, TOTAL)
    dg_in = _padto(in_dst, E_PAD_IN, TOTAL)
    dg_ni = _padto(ni_dst + N_NODE, E_PAD_IN, TOTAL)
    dg_svc = _padto(svc_dst + N_NODE + N_POD, E_PAD_SVC, TOTAL)

    s8_in = _padto((in_src + N_NODE) * T, E_PAD_IN + E_EXTRA, 0)
    s8_ni = _padto(ni_src * T, E_PAD_IN + E_EXTRA, 0)
    s8_svc = _padto((svc_src + N_NODE + N_POD) * T, E_PAD_SVC + E_EXTRA, 0)
    d_in = _padto(in_dst, E_PAD_IN + E_EXTRA, N_NODE)
    d_ni = _padto(ni_dst, E_PAD_IN + E_EXTRA, N_POD)
    d_svc = _padto(svc_dst, E_PAD_SVC + E_EXTRA, N_SVC)
    zhbm = jnp.zeros((ZROWS, F), jnp.float32)

    cnt_src, cnt_dst = _sc_counts(sg_in, sg_ni, sg_svc, dg_in, dg_ni, dg_svc)
    xs = _prescale(xcat, cnt_src[:, :TOTAL])
    agg = _sc_agg(xs.reshape(TOTAL * T, F),
                  s8_in, s8_ni, s8_svc, d_in, d_ni, d_svc, zhbm)

    bias0 = bih0 + bhh0
    bias1 = bih1 + bhh1
    wc0 = jnp.concatenate([Wih0.T, Whh0.T], axis=0)
    wc1 = jnp.concatenate([Wih1.T, Whh1.T], axis=0)

    def run(lo, hi, W, b, R):
        return _conv_lstm(agg[:, lo:hi], cnt_dst[:, lo:hi], W, b,
                          wc0, bias0, wc1, bias1, R)

    out_node = run(0, N_NODE, W_in, b_in, 1000)
    out_pod = run(N_NODE, N_NODE + N_POD, W_ni, b_ni, 1000)
    out_svc = run(N_NODE + N_POD, TOTAL, W_svc, b_svc, 1000)
    return jnp.concatenate([out_node, out_pod, out_svc], axis=0)


# speedup vs baseline: 2.5415x; 1.0797x over previous
import functools

import jax
import jax.numpy as jnp
from jax import lax
from jax.experimental import pallas as pl
from jax.experimental.pallas import tpu as pltpu
from jax.experimental.pallas import tpu_sc as plsc

N_NODE, N_POD, N_SVC = 10000, 30000, 4000
T, F, H = 8, 64, 64
TOTAL = N_NODE + N_POD + N_SVC

NC, NS = 2, 16
KB = 128
KBC = 64
E_PAD_IN = 32768
E_PAD_SVC = 65536
E_EXTRA = KBC
CNT_ROWS = 44032
CNT_PER_TILE = CNT_ROWS // NS
AGG_TAB = 30016
ZROWS = 1888
ZC_A = 344

_mesh = plsc.VectorSubcoreMesh(core_axis_name="c", subcore_axis_name="s",
                               num_cores=NC, num_subcores=NS)
_sc_params = pltpu.CompilerParams(use_tc_tiling_on_sc=False)


@functools.partial(
    pl.kernel,
    out_type=(jax.ShapeDtypeStruct((NC, CNT_ROWS, 16), jnp.float32),
              jax.ShapeDtypeStruct((NC, CNT_ROWS, 16), jnp.float32)),
    mesh=_mesh,
    scratch_types=[
        pltpu.VMEM_SHARED((CNT_ROWS, 16), jnp.float32),
        pltpu.VMEM_SHARED((CNT_ROWS, 16), jnp.float32),
        pltpu.VMEM((ZC_A, 16), jnp.float32),
        pltpu.VMEM((KB, 16), jnp.float32),
        pltpu.VMEM((KB,), jnp.int32),
    ],
    compiler_params=_sc_params,
)
def _sc_counts(sg_in, sg_ni, sg_svc, dg_in, dg_ni, dg_svc,
               cnt_src_out, cnt_dst_out, tab_s, tab_d, zbuf, onesbuf, idxbuf):
    c = lax.axis_index("c")
    s = lax.axis_index("s")
    wid = s * NC + c

    zero16 = jnp.zeros((16,), jnp.float32)
    e0 = jnp.where(lax.iota(jnp.int32, 16) == 0,
                   jnp.float32(1.0), jnp.float32(0.0))

    def fill_z(i, _):
        zbuf[i, :] = zero16
        return 0
    lax.fori_loop(0, ZC_A, fill_z, 0)

    def fill_o(i, _):
        onesbuf[i, :] = e0
        return 0
    lax.fori_loop(0, KB, fill_o, 0)

    r0 = s * CNT_PER_TILE

    def zero_tabs(i, _):
        pltpu.sync_copy(zbuf, tab_s.at[pl.ds(r0 + i * ZC_A, ZC_A)])
        pltpu.sync_copy(zbuf, tab_d.at[pl.ds(r0 + i * ZC_A, ZC_A)])
        return 0
    lax.fori_loop(0, CNT_PER_TILE // ZC_A, zero_tabs, 0)
    plsc.subcore_barrier()

    def scat(arr, tab, nb):
        base = wid * (nb * KB)

        def body(i, _):
            pltpu.sync_copy(arr.at[pl.ds(base + i * KB, KB)], idxbuf)
            pltpu.sync_copy(onesbuf, tab.at[idxbuf], add=True)
            return 0
        lax.fori_loop(0, nb, body, 0)

    scat(sg_in, tab_s, E_PAD_IN // (NC * NS * KB))
    scat(sg_ni, tab_s, E_PAD_IN // (NC * NS * KB))
    scat(sg_svc, tab_s, E_PAD_SVC // (NC * NS * KB))
    scat(dg_in, tab_d, E_PAD_IN // (NC * NS * KB))
    scat(dg_ni, tab_d, E_PAD_IN // (NC * NS * KB))
    scat(dg_svc, tab_d, E_PAD_SVC // (NC * NS * KB))
    plsc.subcore_barrier()

    pltpu.sync_copy(tab_s.at[pl.ds(r0, CNT_PER_TILE)],
                    cnt_src_out.at[c, pl.ds(r0, CNT_PER_TILE)])
    pltpu.sync_copy(tab_d.at[pl.ds(r0, CNT_PER_TILE)],
                    cnt_dst_out.at[c, pl.ds(r0, CNT_PER_TILE)])


@functools.partial(
    pl.kernel,
    out_type=jax.ShapeDtypeStruct((T, TOTAL, F), jnp.float32),
    mesh=_mesh,
    scratch_types=[
        pltpu.VMEM_SHARED((AGG_TAB, F), jnp.float32),
        pltpu.VMEM((KBC, F), jnp.float32),
        pltpu.VMEM((KBC,), jnp.int32),
        pltpu.VMEM((KBC,), jnp.int32),
        pltpu.SemaphoreType.DMA,
        pltpu.VMEM((KBC, F), jnp.float32),
        pltpu.VMEM((KBC,), jnp.int32),
        pltpu.VMEM((KBC,), jnp.int32),
        pltpu.SemaphoreType.DMA,
    ],
    compiler_params=_sc_params,
)
def _sc_agg(xs_flat, s8_in, s8_ni, s8_svc, d_in, d_ni, d_svc, zhbm,
            agg_out, tab, rowA, idxsA, idxdA, semA, rowB, idxsB, idxdB, semB):
    c = lax.axis_index("c")
    s = lax.axis_index("s")
    bufs = ((rowA, idxsA, idxdA, semA), (rowB, idxsB, idxdB, semB))

    sections = (
        (s8_in, d_in, N_NODE, 10240, E_PAD_IN // (NS * KBC), 0, 1, 0, T),
        (s8_ni, d_ni, N_POD, AGG_TAB, E_PAD_IN // (NS * KBC), N_NODE,
         0, 0, 3),
        (s8_ni, d_ni, N_POD, AGG_TAB, E_PAD_IN // (NS * KBC), N_NODE,
         1, 3, T),
        (s8_svc, d_svc, N_SVC, 4096, E_PAD_SVC // (NS * KBC),
         N_NODE + N_POD, 0, 0, T),
    )

    for (srcarr, dstarr, n_dst, tabrows, nb, gbase, core,
         t_lo, t_hi) in sections:
        zpt = tabrows // NS
        cpt = n_dst // NS
        ebase = s * (nb * KBC)

        def start(buf, off, t, srcarr=srcarr, dstarr=dstarr):
            rb, ixs, ixd, sm = buf
            pltpu.sync_copy(srcarr.at[pl.ds(off, KBC)], ixs)
            pltpu.async_copy(xs_flat.at[t].at[ixs], rb, sm)
            pltpu.sync_copy(dstarr.at[pl.ds(off, KBC)], ixd)

        def finish(buf, t):
            rb, ixs, ixd, sm = buf
            pltpu.make_async_copy(xs_flat.at[t].at[ixs], rb, sm).wait()
            pltpu.sync_copy(rb, tab.at[ixd], add=True)

        def drain(buf, t):
            rb, ixs, ixd, sm = buf
            pltpu.make_async_copy(xs_flat.at[t].at[ixs], rb, sm).wait()

        def per_t(t, _, zpt=zpt, cpt=cpt, nb=nb, gbase=gbase, ebase=ebase,
                  start=start, finish=finish, drain=drain):
            pltpu.sync_copy(zhbm.at[pl.ds(0, zpt)],
                            tab.at[pl.ds(s * zpt, zpt)])
            plsc.subcore_barrier()

            start(bufs[0], ebase, t)

            def body(g, _):
                start(bufs[1], ebase + (2 * g + 1) * KBC, t)
                finish(bufs[0], t)
                start(bufs[0], ebase + (2 * g + 2) * KBC, t)
                finish(bufs[1], t)
                return 0
            lax.fori_loop(0, nb // 2, body, 0)
            drain(bufs[0], t)
            plsc.subcore_barrier()

            pltpu.sync_copy(tab.at[pl.ds(s * cpt, cpt)],
                            agg_out.at[t, pl.ds(gbase + s * cpt, cpt)])
            plsc.subcore_barrier()
            return 0

        @pl.when(c == core)
        def _run(per_t=per_t, t_lo=t_lo, t_hi=t_hi):
            lax.fori_loop(t_lo, t_hi, per_t, 0)


def _prescale_body(x_ref, cnt_ref, out_ref):
    deg = cnt_ref[0, :, 0] + cnt_ref[1, :, 0]
    nrm = lax.rsqrt(jnp.maximum(deg, 1.0))
    xsc = x_ref[...] * nrm[:, None]
    for t in range(T):
        out_ref[t] = xsc[:, t * F:(t + 1) * F]


def _prescale(xcat, cnt_src, R=400):
    n = xcat.shape[0]
    return pl.pallas_call(
        _prescale_body,
        grid=(n // R,),
        in_specs=[
            pl.BlockSpec((R, T * F), lambda i: (i, 0)),
            pl.BlockSpec((NC, R, 16), lambda i: (0, i, 0)),
        ],
        out_specs=pl.BlockSpec((T, R, F), lambda i: (0, i, 0)),
        out_shape=jax.ShapeDtypeStruct((T, n, F), jnp.float32),
    )(xcat, cnt_src)


def _conv_lstm_body(agg_ref, cnt_ref, W_ref, b_ref,
                    wc0_ref, bias0_ref, wc1_ref, bias1_ref,
                    out_ref):
    R = agg_ref.shape[1]
    deg = cnt_ref[0, :, 0] + cnt_ref[1, :, 0]
    nrm = lax.rsqrt(jnp.maximum(deg, 1.0))

    def lrelu(v):
        return jnp.where(v > 0, v, 0.01 * v)

    bf = jnp.bfloat16
    xs = []
    for t in range(T):
        y = jnp.dot((agg_ref[t] * nrm[:, None]).astype(bf),
                    W_ref[0, t].astype(bf),
                    preferred_element_type=jnp.float32) + b_ref[0, t]
        xs.append(lrelu(y))

    def lstm(x_list, wc, bias):
        wcb = wc.astype(bf)
        h = jnp.zeros((R, H), jnp.float32)
        cc = jnp.zeros((R, H), jnp.float32)
        outs = []
        for t in range(T):
            xh = jnp.concatenate([x_list[t], h], axis=1).astype(bf)
            g = jnp.dot(xh, wcb, preferred_element_type=jnp.float32) + bias
            i = jax.nn.sigmoid(g[:, 0 * H:1 * H])
            f = jax.nn.sigmoid(g[:, 1 * H:2 * H])
            gg = jnp.tanh(g[:, 2 * H:3 * H])
            o = jax.nn.sigmoid(g[:, 3 * H:4 * H])
            cc = f * cc + i * gg
            h = o * jnp.tanh(cc)
            outs.append(h)
        return outs

    h1 = lstm(xs, wc0_ref[...], bias0_ref[...])
    h2 = lstm(h1, wc1_ref[...], bias1_ref[...])
    out_ref[...] = jnp.stack(h2, axis=1)


def _conv_lstm(agg, cnt, Wall, ball, wc0, bias0, wc1, bias1, R):
    def typ(i):
        return ((i >= N_NODE // R).astype(jnp.int32)
                + (i >= (N_NODE + N_POD) // R).astype(jnp.int32))

    return pl.pallas_call(
        _conv_lstm_body,
        grid=(TOTAL // R,),
        in_specs=[
            pl.BlockSpec((T, R, F), lambda i: (0, i, 0)),
            pl.BlockSpec((NC, R, 16), lambda i: (0, i, 0)),
            pl.BlockSpec((1, T, F, H), lambda i: (typ(i), 0, 0, 0)),
            pl.BlockSpec((1, T, 1, H), lambda i: (typ(i), 0, 0, 0)),
            pl.BlockSpec((2 * H, 4 * H), lambda i: (0, 0)),
            pl.BlockSpec((1, 4 * H), lambda i: (0, 0)),
            pl.BlockSpec((2 * H, 4 * H), lambda i: (0, 0)),
            pl.BlockSpec((1, 4 * H), lambda i: (0, 0)),
        ],
        out_specs=pl.BlockSpec((R, T, H), lambda i: (i, 0, 0)),
        out_shape=jax.ShapeDtypeStruct((TOTAL, T, H), jnp.float32),
    )(agg, cnt, Wall, ball, wc0, bias0.reshape(1, 4 * H),
      wc1, bias1.reshape(1, 4 * H))


def _padto(a, n, fill):
    return jnp.concatenate(
        [a.astype(jnp.int32), jnp.full((n - a.shape[0],), fill, jnp.int32)])


def kernel(node_feat, pod_feat, svc_feat, svc_src, svc_dst, in_src, in_dst,
           ni_src, ni_dst, W_svc, b_svc, W_in, b_in, W_ni, b_ni,
           Wih0, Whh0, bih0, bhh0, Wih1, Whh1, bih1, bhh1):
    xcat = jnp.concatenate([node_feat.reshape(N_NODE, T * F),
                            pod_feat.reshape(N_POD, T * F),
                            svc_feat.reshape(N_SVC, T * F)], axis=0)

    sg_in = _padto(in_src + N_NODE, E_PAD_IN, TOTAL)
    sg_ni = _padto(ni_src, E_PAD_IN, TOTAL)
    sg_svc = _padto(svc_src + N_NODE + N_POD, E_PAD_SVC, TOTAL)
    dg_in = _padto(in_dst, E_PAD_IN, TOTAL)
    dg_ni = _padto(ni_dst + N_NODE, E_PAD_IN, TOTAL)
    dg_svc = _padto(svc_dst + N_NODE + N_POD, E_PAD_SVC, TOTAL)

    s8_in = _padto(in_src + N_NODE, E_PAD_IN + E_EXTRA, 0)
    s8_ni = _padto(ni_src, E_PAD_IN + E_EXTRA, 0)
    s8_svc = _padto(svc_src + N_NODE + N_POD, E_PAD_SVC + E_EXTRA, 0)
    d_in = _padto(in_dst, E_PAD_IN + E_EXTRA, N_NODE)
    d_ni = _padto(ni_dst, E_PAD_IN + E_EXTRA, N_POD)
    d_svc = _padto(svc_dst, E_PAD_SVC + E_EXTRA, N_SVC)
    zhbm = jnp.zeros((ZROWS, F), jnp.float32)

    cnt_src, cnt_dst = _sc_counts(sg_in, sg_ni, sg_svc, dg_in, dg_ni, dg_svc)
    xs = _prescale(xcat, cnt_src)
    agg = _sc_agg(xs, s8_in, s8_ni, s8_svc, d_in, d_ni, d_svc, zhbm)

    bias0 = bih0 + bhh0
    bias1 = bih1 + bhh1
    wc0 = jnp.concatenate([Wih0.T, Whh0.T], axis=0)
    wc1 = jnp.concatenate([Wih1.T, Whh1.T], axis=0)
    Wall = jnp.stack([W_in, W_ni, W_svc])
    ball = jnp.stack([b_in, b_ni, b_svc]).reshape(3, T, 1, H)

    return _conv_lstm(agg, cnt_dst, Wall, ball,
                      wc0, bias0, wc1, bias1, 1000)
